# Initial kernel scaffold; baseline (speedup 1.0000x reference)
import jax
import jax.numpy as jnp
from jax.experimental import pallas as pl


def kernel(features, labels, epoch, running_mean_last_epoch, running_var_last_epoch, smoothed_mean_last_epoch, smoothed_var_last_epoch, num_samples_tracked):
    raise NotImplementedError("write your pallas kernel here")


# speedup vs baseline: 5.8434x; 5.8434x over previous
import jax
import jax.numpy as jnp
from jax.experimental import pallas as pl
from jax.experimental.pallas import tpu as pltpu

N = 500000
D = 64
BUCKET_NUM = 100
BUCKET_START = 3
START_SMOOTH = 1
EPSILON = 1e-05
NB = BUCKET_NUM - BUCKET_START
NBP = 128
R = 4000


def _prep_body(ep_ref, nst_ref, rm_ref, rv_ref, sm_ref, sv_ref, comb_ref):
    nst = nst_ref[...]
    mean_nst = jnp.sum(nst) / float(NB)
    alpha = jnp.exp(-nst / (mean_nst + EPSILON))
    rm = rm_ref[...]
    rv = rv_ref[...]
    m2 = (1.0 - alpha) * rm + alpha * sm_ref[...]
    v2 = (1.0 - alpha) * rv + alpha * sv_ref[...]
    scale = jnp.sqrt(jnp.clip(v2 / rv, 0.1, 10.0))
    bias = m2 - rm * scale
    row = jax.lax.broadcasted_iota(jnp.int32, (NBP, D), 0)
    valid = row < NB
    use_id = ep_ref[0, 0] < START_SMOOTH
    scale = jnp.where(valid, jnp.where(use_id, 1.0, scale), 0.0)
    bias = jnp.where(valid, jnp.where(use_id, 0.0, bias), 0.0)
    comb_ref[:, :D] = scale
    comb_ref[:, D:] = bias


def _main_body(lab_ref, f_ref, comb_ref, out_ref):
    lab = lab_ref[...]
    b = jnp.clip(lab, BUCKET_START, BUCKET_NUM - 1) - BUCKET_START
    cols = jax.lax.broadcasted_iota(jnp.int32, (R, NBP), 1)
    onehot = (b == cols).astype(jnp.float32)
    g = jnp.dot(onehot, comb_ref[...], preferred_element_type=jnp.float32)
    out_ref[...] = f_ref[...] * g[:, :D] + g[:, D:]


def kernel(features, labels, epoch, running_mean_last_epoch, running_var_last_epoch,
           smoothed_mean_last_epoch, smoothed_var_last_epoch, num_samples_tracked):
    ep = jnp.asarray(epoch, jnp.int32).reshape(1, 1)
    pad = lambda a: jnp.pad(a, ((0, NBP - NB), (0, 0)))
    nst2 = pad(num_samples_tracked.reshape(NB, 1))

    comb = pl.pallas_call(
        _prep_body,
        out_shape=jax.ShapeDtypeStruct((NBP, 2 * D), jnp.float32),
    )(ep, nst2, pad(running_mean_last_epoch), pad(running_var_last_epoch),
      pad(smoothed_mean_last_epoch), pad(smoothed_var_last_epoch))

    out = pl.pallas_call(
        _main_body,
        grid=(N // R,),
        in_specs=[
            pl.BlockSpec((R, 1), lambda i: (i, 0)),
            pl.BlockSpec((R, D), lambda i: (i, 0)),
            pl.BlockSpec((NBP, 2 * D), lambda i: (0, 0)),
        ],
        out_specs=pl.BlockSpec((R, D), lambda i: (i, 0)),
        out_shape=jax.ShapeDtypeStruct((N, D), jnp.float32),
        compiler_params=pltpu.CompilerParams(
            dimension_semantics=("arbitrary",),
        ),
    )(labels, features, comb)
    return out
